# baseline (speedup 1.0000x reference)
import functools

import jax
import jax.numpy as jnp
from jax import lax
from jax.experimental import pallas as pl
from jax.experimental.pallas import tpu as pltpu
from jax.experimental.pallas import tpu_sc as plsc

N = 10000
NPAD = 10240
E = 160000
D_IN = 256
D_HID = 512
D_OUT = 256

NC = 2
NS = 16
B = 80
EPT = E // NS
NB = EPT // B
RPT = NPAD // NS


def _make_sc_agg(n_chunks: int, with_counts: bool):
    n_passes = n_chunks // 2
    out_type = [jax.ShapeDtypeStruct((n_chunks * NPAD, 128), jnp.float32)]
    if with_counts:
        out_type.append(jax.ShapeDtypeStruct((NPAD,), jnp.float32))
    mesh = plsc.VectorSubcoreMesh(core_axis_name="c", subcore_axis_name="s")
    scratch = [
        pltpu.VMEM((2, B), jnp.int32),
        pltpu.VMEM((2, B), jnp.int32),
        pltpu.VMEM((2, B), jnp.int32),
        pltpu.VMEM((2, B), jnp.int32),
        pltpu.VMEM((2, B, 128), jnp.float32),
        pltpu.VMEM((B,), jnp.float32),
        pltpu.VMEM_SHARED((NPAD, 128), jnp.float32),
        pltpu.VMEM_SHARED((NPAD,), jnp.float32),
        pltpu.SemaphoreType.DMA,
        pltpu.SemaphoreType.DMA,
        pltpu.SemaphoreType.DMA,
        pltpu.SemaphoreType.DMA,
        pltpu.SemaphoreType.DMA,
        pltpu.SemaphoreType.DMA,
    ]

    def body(src_hbm, dst_hbm, table_hbm, z2d_hbm, z1d_hbm, *rest):
        if with_counts:
            out_hbm, cnt_hbm = rest[0], rest[1]
            rest = rest[2:]
        else:
            out_hbm = rest[0]
            rest = rest[1:]
        (sidx_v, didx_v, ebuf_v, dbuf_v, rows_v, ones_v, acc_sh, cnt_sh,
         gsem0, gsem1, isem0, isem1, ssem0, ssem1) = rest
        gsems = (gsem0, gsem1)
        isems = (isem0, isem1)
        ssems = (ssem0, ssem1)
        c = lax.axis_index("c")
        s = lax.axis_index("s")
        r0 = s * RPT
        ebase = s * EPT

        if with_counts:
            for j in range(B // 16):
                ones_v[pl.ds(j * 16, 16)] = jnp.full((16,), 1.0, jnp.float32)

        def start_idx(i, b):
            e0 = ebase + i * B
            pltpu.async_copy(src_hbm.at[pl.ds(e0, B)], sidx_v.at[b], isems[b])
            pltpu.async_copy(dst_hbm.at[pl.ds(e0, B)], didx_v.at[b], isems[b])

        def wait_idx(b):
            pltpu.make_async_copy(src_hbm.at[pl.ds(0, B)], sidx_v.at[b],
                                  isems[b]).wait()
            pltpu.make_async_copy(dst_hbm.at[pl.ds(0, B)], didx_v.at[b],
                                  isems[b]).wait()

        for p in range(n_passes):
            ch = 2 * p + c
            off = ch * NPAD

            def start_gather(b):
                for j in range(B // 16):
                    sl = pl.ds(j * 16, 16)
                    ebuf_v[b, sl] = sidx_v[b, sl] + off
                pltpu.async_copy(table_hbm.at[ebuf_v.at[b]], rows_v.at[b],
                                 gsems[b])

            def wait_gather(b):
                pltpu.make_async_copy(table_hbm.at[ebuf_v.at[b]],
                                      rows_v.at[b], gsems[b]).wait()

            def start_scatter(b):
                for j in range(B // 16):
                    sl = pl.ds(j * 16, 16)
                    dbuf_v[b, sl] = didx_v[b, sl]
                pltpu.async_copy(rows_v.at[b], acc_sh.at[dbuf_v.at[b]],
                                 ssems[b], add=True)
                if with_counts and p == 0:
                    @pl.when(c == 0)
                    def _():
                        pltpu.sync_copy(ones_v, cnt_sh.at[dbuf_v.at[b]],
                                        add=True)

            def wait_scatter(b):
                pltpu.make_async_copy(rows_v.at[b], acc_sh.at[dbuf_v.at[b]],
                                      ssems[b]).wait()

            pltpu.sync_copy(z2d_hbm, acc_sh.at[pl.ds(r0, RPT)])
            if with_counts and p == 0:
                @pl.when(c == 0)
                def _():
                    pltpu.sync_copy(z1d_hbm, cnt_sh.at[pl.ds(r0, RPT)])
            plsc.subcore_barrier()

            start_idx(0, 0)
            wait_idx(0)
            start_gather(0)
            start_idx(1, 1)

            def step(i, b, nb, first=False):
                wait_idx(nb)
                if not first:
                    wait_scatter(nb)
                start_gather(nb)
                wait_gather(b)
                start_scatter(b)
                start_idx(jnp.minimum(i + 2, NB - 1), b)

            step(0, 0, 1, first=True)

            def pair(k, carry):
                step(2 * k + 1, 1, 0)
                step(2 * k + 2, 0, 1)
                return carry

            lax.fori_loop(0, (NB - 3) // 2, pair, 0)
            step(NB - 2, 1, 0)
            wait_gather(0)
            start_scatter(0)
            wait_scatter(1)
            wait_scatter(0)
            wait_idx(1)

            plsc.subcore_barrier()
            pltpu.sync_copy(acc_sh.at[pl.ds(r0, RPT)],
                            out_hbm.at[pl.ds(off + r0, RPT)])
            if with_counts and p == 0:
                @pl.when(c == 0)
                def _():
                    pltpu.sync_copy(cnt_sh.at[pl.ds(r0, RPT)],
                                    cnt_hbm.at[pl.ds(r0, RPT)])
            if p + 1 < n_passes:
                plsc.subcore_barrier()

    return pl.kernel(body, out_type=out_type, mesh=mesh, scratch_types=scratch)


R = 1280
GRID = NPAD // R


def _root_body(x_ref, wr_ref, b_ref, out_ref):
    out_ref[...] = jnp.dot(x_ref[...], wr_ref[...],
                           preferred_element_type=jnp.float32) + b_ref[...]


def _tc1_body(agg_ref, cnt_ref, xr_ref, wl_ref, out_ref):
    agg = jnp.concatenate([agg_ref[0], agg_ref[1]], axis=-1)
    inv = 1.0 / jnp.maximum(cnt_ref[...], 1.0)
    h = jnp.dot(agg * inv, wl_ref[...], preferred_element_type=jnp.float32)
    h += xr_ref[...]
    h = jnp.maximum(h, 0.0)
    for ch in range(4):
        out_ref[ch] = h[:, 128 * ch:128 * (ch + 1)]


def _tc2_body(agg_ref, cnt_ref, hr_ref, wl_ref, wfc_ref, bfc_ref, out_ref):
    agg = jnp.concatenate([agg_ref[i] for i in range(4)], axis=-1)
    inv = 1.0 / jnp.maximum(cnt_ref[...], 1.0)
    h = jnp.dot(agg * inv, wl_ref[...], preferred_element_type=jnp.float32)
    h += hr_ref[...]
    h = jnp.maximum(h, 0.0)
    out_ref[...] = jnp.dot(h, wfc_ref[...],
                           preferred_element_type=jnp.float32) + bfc_ref[...]


def _full(shape):
    return pl.BlockSpec(shape, lambda i: tuple(0 for _ in shape))


_root1 = pl.pallas_call(
    _root_body,
    grid=(GRID,),
    in_specs=[
        pl.BlockSpec((R, D_IN), lambda i: (i, 0)),
        _full((D_IN, D_HID)),
        _full((1, D_HID)),
    ],
    out_specs=pl.BlockSpec((R, D_HID), lambda i: (i, 0)),
    out_shape=jax.ShapeDtypeStruct((NPAD, D_HID), jnp.float32),
)


def _root2_body(h1_ref, wr_ref, b_ref, out_ref):
    h1 = jnp.concatenate([h1_ref[i] for i in range(4)], axis=-1)
    out_ref[...] = jnp.dot(h1, wr_ref[...],
                           preferred_element_type=jnp.float32) + b_ref[...]


_root2 = pl.pallas_call(
    _root2_body,
    grid=(GRID,),
    in_specs=[
        pl.BlockSpec((4, R, 128), lambda i: (0, i, 0)),
        _full((D_HID, D_HID)),
        _full((1, D_HID)),
    ],
    out_specs=pl.BlockSpec((R, D_HID), lambda i: (i, 0)),
    out_shape=jax.ShapeDtypeStruct((NPAD, D_HID), jnp.float32),
)

_tc1 = pl.pallas_call(
    _tc1_body,
    grid=(GRID,),
    in_specs=[
        pl.BlockSpec((2, R, 128), lambda i: (0, i, 0)),
        pl.BlockSpec((R, 1), lambda i: (i, 0)),
        pl.BlockSpec((R, D_HID), lambda i: (i, 0)),
        _full((D_IN, D_HID)),
    ],
    out_specs=pl.BlockSpec((4, R, 128), lambda i: (0, i, 0)),
    out_shape=jax.ShapeDtypeStruct((4, NPAD, 128), jnp.float32),
)

_tc2 = pl.pallas_call(
    _tc2_body,
    grid=(GRID,),
    in_specs=[
        pl.BlockSpec((4, R, 128), lambda i: (0, i, 0)),
        pl.BlockSpec((R, 1), lambda i: (i, 0)),
        pl.BlockSpec((R, D_HID), lambda i: (i, 0)),
        _full((D_HID, D_HID)),
        _full((D_HID, D_OUT)),
        _full((1, D_OUT)),
    ],
    out_specs=pl.BlockSpec((R, D_OUT), lambda i: (i, 0)),
    out_shape=jax.ShapeDtypeStruct((NPAD, D_OUT), jnp.float32),
)

_sc_agg2 = _make_sc_agg(2, with_counts=True)
_sc_agg4 = _make_sc_agg(4, with_counts=False)


def kernel(x, edge_index, W1l, b1, W1r, W2l, b2, W2r, Wfc, bfc):
    src = edge_index[0].astype(jnp.int32)
    dst = edge_index[1].astype(jnp.int32)
    xp = jnp.pad(x, ((0, NPAD - N), (0, 0)))
    xc = xp.reshape(NPAD, 2, 128).transpose(1, 0, 2).reshape(2 * NPAD, 128)
    z2d = jnp.zeros((RPT, 128), jnp.float32)
    z1d = jnp.zeros((RPT,), jnp.float32)

    agg1, cnt = _sc_agg2(src, dst, xc, z2d, z1d)
    xr = _root1(xp, W1r.T, b1.reshape(1, D_HID))
    cnt2 = cnt.reshape(NPAD, 1)
    h1c = _tc1(agg1.reshape(2, NPAD, 128), cnt2, xr, W1l.T)
    agg2, = _sc_agg4(src, dst, h1c.reshape(4 * NPAD, 128), z2d, z1d)
    hr = _root2(h1c, W2r.T, b2.reshape(1, D_HID))
    outp = _tc2(agg2.reshape(4, NPAD, 128), cnt2, hr,
                W2l.T, Wfc.T, bfc.reshape(1, D_OUT))
    return outp[:N]


# speedup vs baseline: 1.0549x; 1.0549x over previous
import functools

import jax
import jax.numpy as jnp
from jax import lax
from jax.experimental import pallas as pl
from jax.experimental.pallas import tpu as pltpu
from jax.experimental.pallas import tpu_sc as plsc

N = 10000
NPAD = 10240
E = 160000
D_IN = 256
D_HID = 512
D_OUT = 256

NC = 2
NS = 16
B = 80
EPT = E // NS
NB = EPT // B
RPT = NPAD // NS


def _make_sc_agg(n_chunks: int, with_counts: bool):
    n_passes = n_chunks // 2
    out_type = [jax.ShapeDtypeStruct((n_chunks * NPAD, 128), jnp.float32)]
    if with_counts:
        out_type.append(jax.ShapeDtypeStruct((NPAD,), jnp.float32))
    mesh = plsc.VectorSubcoreMesh(core_axis_name="c", subcore_axis_name="s")
    scratch = [
        pltpu.VMEM((2, B), jnp.int32),
        pltpu.VMEM((2, B), jnp.int32),
        pltpu.VMEM((2, B), jnp.int32),
        pltpu.VMEM((2, B), jnp.int32),
        pltpu.VMEM((2, B, 128), jnp.float32),
        pltpu.VMEM((B,), jnp.float32),
        pltpu.VMEM_SHARED((NPAD, 128), jnp.float32),
        pltpu.VMEM_SHARED((NPAD,), jnp.float32),
        pltpu.SemaphoreType.DMA,
        pltpu.SemaphoreType.DMA,
        pltpu.SemaphoreType.DMA,
        pltpu.SemaphoreType.DMA,
        pltpu.SemaphoreType.DMA,
        pltpu.SemaphoreType.DMA,
    ]

    def body(src_hbm, dst_hbm, table_hbm, z2d_hbm, z1d_hbm, *rest):
        if with_counts:
            out_hbm, cnt_hbm = rest[0], rest[1]
            rest = rest[2:]
        else:
            out_hbm = rest[0]
            rest = rest[1:]
        (sidx_v, didx_v, ebuf_v, dbuf_v, rows_v, ones_v, acc_sh, cnt_sh,
         gsem0, gsem1, isem0, isem1, ssem0, ssem1) = rest
        gsems = (gsem0, gsem1)
        isems = (isem0, isem1)
        ssems = (ssem0, ssem1)
        c = lax.axis_index("c")
        s = lax.axis_index("s")
        r0 = s * RPT
        ebase = s * EPT

        if with_counts:
            for j in range(B // 16):
                ones_v[pl.ds(j * 16, 16)] = jnp.full((16,), 1.0, jnp.float32)

        def start_idx(i, b):
            e0 = ebase + i * B
            pltpu.async_copy(src_hbm.at[pl.ds(e0, B)], sidx_v.at[b], isems[b])
            pltpu.async_copy(dst_hbm.at[pl.ds(e0, B)], didx_v.at[b], isems[b])

        def wait_idx(b):
            pltpu.make_async_copy(src_hbm.at[pl.ds(0, B)], sidx_v.at[b],
                                  isems[b]).wait()
            pltpu.make_async_copy(dst_hbm.at[pl.ds(0, B)], didx_v.at[b],
                                  isems[b]).wait()

        for p in range(n_passes):
            ch = 2 * p + c
            off = ch * NPAD

            def start_gather(b):
                for j in range(B // 16):
                    sl = pl.ds(j * 16, 16)
                    ebuf_v[b, sl] = sidx_v[b, sl] + off
                pltpu.async_copy(table_hbm.at[ebuf_v.at[b]], rows_v.at[b],
                                 gsems[b])

            def wait_gather(b):
                pltpu.make_async_copy(table_hbm.at[ebuf_v.at[b]],
                                      rows_v.at[b], gsems[b]).wait()

            def start_scatter(b):
                for j in range(B // 16):
                    sl = pl.ds(j * 16, 16)
                    dbuf_v[b, sl] = didx_v[b, sl]
                pltpu.async_copy(rows_v.at[b], acc_sh.at[dbuf_v.at[b]],
                                 ssems[b], add=True)
                if with_counts and p == 0:
                    @pl.when(c == 0)
                    def _():
                        pltpu.sync_copy(ones_v, cnt_sh.at[dbuf_v.at[b]],
                                        add=True)

            def wait_scatter(b):
                pltpu.make_async_copy(rows_v.at[b], acc_sh.at[dbuf_v.at[b]],
                                      ssems[b]).wait()

            def zrow(i, carry):
                for j in range(8):
                    rows_v[0, i, pl.ds(j * 16, 16)] = jnp.zeros((16,),
                                                                jnp.float32)
                return carry

            lax.fori_loop(0, B, zrow, 0)
            for k in range(RPT // B):
                pltpu.sync_copy(rows_v.at[0],
                                acc_sh.at[pl.ds(r0 + k * B, B)])
            if with_counts and p == 0:
                @pl.when(c == 0)
                def _():
                    pltpu.sync_copy(z1d_hbm, cnt_sh.at[pl.ds(r0, RPT)])
            plsc.subcore_barrier()

            start_idx(0, 0)
            wait_idx(0)
            start_gather(0)
            start_idx(1, 1)

            def step(i, b, nb, first=False):
                wait_idx(nb)
                if not first:
                    wait_scatter(nb)
                start_gather(nb)
                wait_gather(b)
                start_scatter(b)
                start_idx(jnp.minimum(i + 2, NB - 1), b)

            step(0, 0, 1, first=True)

            def pair(k, carry):
                step(2 * k + 1, 1, 0)
                step(2 * k + 2, 0, 1)
                return carry

            lax.fori_loop(0, (NB - 3) // 2, pair, 0)
            step(NB - 2, 1, 0)
            wait_gather(0)
            start_scatter(0)
            wait_scatter(1)
            wait_scatter(0)
            wait_idx(1)

            plsc.subcore_barrier()
            pltpu.sync_copy(acc_sh.at[pl.ds(r0, RPT)],
                            out_hbm.at[pl.ds(off + r0, RPT)])
            if with_counts and p == 0:
                @pl.when(c == 0)
                def _():
                    pltpu.sync_copy(cnt_sh.at[pl.ds(r0, RPT)],
                                    cnt_hbm.at[pl.ds(r0, RPT)])
            if p + 1 < n_passes:
                plsc.subcore_barrier()

    return pl.kernel(body, out_type=out_type, mesh=mesh, scratch_types=scratch)


R = 1280
GRID = NPAD // R


def _tc1_body(agg_ref, cnt_ref, x_ref, wl_ref, wr_ref, b_ref, out_ref):
    agg = jnp.concatenate([agg_ref[0], agg_ref[1]], axis=-1)
    inv = 1.0 / jnp.maximum(cnt_ref[...], 1.0)
    h = jnp.dot(agg * inv, wl_ref[...], preferred_element_type=jnp.float32)
    h += jnp.dot(x_ref[...], wr_ref[...], preferred_element_type=jnp.float32)
    h += b_ref[...]
    h = jnp.maximum(h, 0.0)
    for ch in range(4):
        out_ref[ch] = h[:, 128 * ch:128 * (ch + 1)]


def _tc2_body(agg_ref, cnt_ref, h1_ref, wl_ref, wr_ref, b_ref, wfc_ref,
              bfc_ref, out_ref):
    agg = jnp.concatenate([agg_ref[i] for i in range(4)], axis=-1)
    h1 = jnp.concatenate([h1_ref[i] for i in range(4)], axis=-1)
    inv = 1.0 / jnp.maximum(cnt_ref[...], 1.0)
    h = jnp.dot(agg * inv, wl_ref[...], preferred_element_type=jnp.float32)
    h += jnp.dot(h1, wr_ref[...], preferred_element_type=jnp.float32)
    h += b_ref[...]
    h = jnp.maximum(h, 0.0)
    out_ref[...] = jnp.dot(h, wfc_ref[...],
                           preferred_element_type=jnp.float32) + bfc_ref[...]


def _full(shape):
    return pl.BlockSpec(shape, lambda i: tuple(0 for _ in shape))


_tc1 = pl.pallas_call(
    _tc1_body,
    grid=(GRID,),
    in_specs=[
        pl.BlockSpec((2, R, 128), lambda i: (0, i, 0)),
        pl.BlockSpec((R, 1), lambda i: (i, 0)),
        pl.BlockSpec((R, D_IN), lambda i: (i, 0)),
        _full((D_IN, D_HID)),
        _full((D_IN, D_HID)),
        _full((1, D_HID)),
    ],
    out_specs=pl.BlockSpec((4, R, 128), lambda i: (0, i, 0)),
    out_shape=jax.ShapeDtypeStruct((4, NPAD, 128), jnp.float32),
)

_tc2 = pl.pallas_call(
    _tc2_body,
    grid=(GRID,),
    in_specs=[
        pl.BlockSpec((4, R, 128), lambda i: (0, i, 0)),
        pl.BlockSpec((R, 1), lambda i: (i, 0)),
        pl.BlockSpec((4, R, 128), lambda i: (0, i, 0)),
        _full((D_HID, D_HID)),
        _full((D_HID, D_HID)),
        _full((1, D_HID)),
        _full((D_HID, D_OUT)),
        _full((1, D_OUT)),
    ],
    out_specs=pl.BlockSpec((R, D_OUT), lambda i: (i, 0)),
    out_shape=jax.ShapeDtypeStruct((NPAD, D_OUT), jnp.float32),
)

_sc_agg2 = _make_sc_agg(2, with_counts=True)
_sc_agg4 = _make_sc_agg(4, with_counts=False)


def kernel(x, edge_index, W1l, b1, W1r, W2l, b2, W2r, Wfc, bfc):
    src = edge_index[0].astype(jnp.int32)
    dst = edge_index[1].astype(jnp.int32)
    xp = jnp.pad(x, ((0, NPAD - N), (0, 0)))
    xc = xp.reshape(NPAD, 2, 128).transpose(1, 0, 2).reshape(2 * NPAD, 128)
    z2d = jnp.zeros((RPT, 128), jnp.float32)
    z1d = jnp.zeros((RPT,), jnp.float32)

    agg1, cnt = _sc_agg2(src, dst, xc, z2d, z1d)
    cnt2 = cnt.reshape(NPAD, 1)
    h1c = _tc1(agg1.reshape(2, NPAD, 128), cnt2, xp,
               W1l.T, W1r.T, b1.reshape(1, D_HID))
    agg2, = _sc_agg4(src, dst, h1c.reshape(4 * NPAD, 128), z2d, z1d)
    outp = _tc2(agg2.reshape(4, NPAD, 128), cnt2, h1c,
                W2l.T, W2r.T, b2.reshape(1, D_HID),
                Wfc.T, bfc.reshape(1, D_OUT))
    return outp[:N]


# speedup vs baseline: 1.0661x; 1.0106x over previous
import functools

import jax
import jax.numpy as jnp
from jax import lax
from jax.experimental import pallas as pl
from jax.experimental.pallas import tpu as pltpu
from jax.experimental.pallas import tpu_sc as plsc

N = 10000
NPAD = 10240
E = 160000
D_IN = 256
D_HID = 512
D_OUT = 256

NC = 2
NS = 16
B = 80
EPT = E // NS
NB = EPT // B
RPT = NPAD // NS


def _make_sc_agg(n_chunks: int, with_counts: bool):
    n_passes = n_chunks // 2
    out_type = [jax.ShapeDtypeStruct((n_chunks * NPAD, 128), jnp.float32)]
    if with_counts:
        out_type.append(jax.ShapeDtypeStruct((NPAD,), jnp.float32))
    mesh = plsc.VectorSubcoreMesh(core_axis_name="c", subcore_axis_name="s")
    scratch = [
        pltpu.VMEM((2, B), jnp.int32),
        pltpu.VMEM((2, B), jnp.int32),
        pltpu.VMEM((2, B), jnp.int32),
        pltpu.VMEM((2, B), jnp.int32),
        pltpu.VMEM((2, B, 128), jnp.float32),
        pltpu.VMEM((B,), jnp.float32),
        pltpu.VMEM((B,), jnp.float32),
        pltpu.VMEM_SHARED((NPAD, 128), jnp.float32),
        pltpu.VMEM_SHARED((NPAD,), jnp.float32),
        pltpu.SemaphoreType.DMA,
        pltpu.SemaphoreType.DMA,
        pltpu.SemaphoreType.DMA,
        pltpu.SemaphoreType.DMA,
        pltpu.SemaphoreType.DMA,
        pltpu.SemaphoreType.DMA,
        pltpu.SemaphoreType.DMA,
        pltpu.SemaphoreType.DMA,
    ]

    def body(src_hbm, dst_hbm, table_hbm, z2d_hbm, z1d_hbm, *rest):
        if with_counts:
            out_hbm, cnt_hbm = rest[0], rest[1]
            rest = rest[2:]
        else:
            out_hbm = rest[0]
            rest = rest[1:]
        (sidx_v, didx_v, ebuf_v, dbuf_v, rows_v, ones_v, zc_v, acc_sh,
         cnt_sh, gsem0, gsem1, isem0, isem1, ssem0, ssem1, csem0,
         csem1) = rest
        gsems = (gsem0, gsem1)
        isems = (isem0, isem1)
        ssems = (ssem0, ssem1)
        csems = (csem0, csem1)
        c = lax.axis_index("c")
        s = lax.axis_index("s")
        r0 = s * RPT
        ebase = s * EPT

        if with_counts:
            for j in range(B // 16):
                sl = pl.ds(j * 16, 16)
                ones_v[sl] = jnp.full((16,), 1.0, jnp.float32)
                zc_v[sl] = jnp.zeros((16,), jnp.float32)
                dbuf_v[0, sl] = jnp.zeros((16,), jnp.int32)
                dbuf_v[1, sl] = jnp.zeros((16,), jnp.int32)

        def start_idx(i, b):
            e0 = ebase + i * B
            pltpu.async_copy(src_hbm.at[pl.ds(e0, B)], sidx_v.at[b], isems[b])
            pltpu.async_copy(dst_hbm.at[pl.ds(e0, B)], didx_v.at[b], isems[b])

        def wait_idx(b):
            pltpu.make_async_copy(src_hbm.at[pl.ds(0, B)], sidx_v.at[b],
                                  isems[b]).wait()
            pltpu.make_async_copy(dst_hbm.at[pl.ds(0, B)], didx_v.at[b],
                                  isems[b]).wait()

        for p in range(n_passes):
            ch = 2 * p + c
            off = ch * NPAD

            def start_gather(b):
                for j in range(B // 16):
                    sl = pl.ds(j * 16, 16)
                    ebuf_v[b, sl] = sidx_v[b, sl] + off
                pltpu.async_copy(table_hbm.at[ebuf_v.at[b]], rows_v.at[b],
                                 gsems[b])

            def wait_gather(b):
                pltpu.make_async_copy(table_hbm.at[ebuf_v.at[b]],
                                      rows_v.at[b], gsems[b]).wait()

            def start_counts(b, vals):
                pltpu.async_copy(vals, cnt_sh.at[dbuf_v.at[b]], csems[b],
                                 add=True)

            def wait_counts(b):
                pltpu.make_async_copy(ones_v, cnt_sh.at[dbuf_v.at[b]],
                                      csems[b]).wait()

            def start_scatter(b):
                if with_counts and p == 0:
                    @pl.when(c == 0)
                    def _():
                        wait_counts(b)
                for j in range(B // 16):
                    sl = pl.ds(j * 16, 16)
                    dbuf_v[b, sl] = didx_v[b, sl]
                pltpu.async_copy(rows_v.at[b], acc_sh.at[dbuf_v.at[b]],
                                 ssems[b], add=True)
                if with_counts and p == 0:
                    @pl.when(c == 0)
                    def _():
                        start_counts(b, ones_v)

            def wait_scatter(b):
                pltpu.make_async_copy(rows_v.at[b], acc_sh.at[dbuf_v.at[b]],
                                      ssems[b]).wait()

            def zrow(i, carry):
                for j in range(8):
                    rows_v[0, i, pl.ds(j * 16, 16)] = jnp.zeros((16,),
                                                                jnp.float32)
                return carry

            lax.fori_loop(0, B, zrow, 0)
            for k in range(RPT // B):
                pltpu.sync_copy(rows_v.at[0],
                                acc_sh.at[pl.ds(r0 + k * B, B)])
            if with_counts and p == 0:
                @pl.when(c == 0)
                def _():
                    pltpu.sync_copy(z1d_hbm, cnt_sh.at[pl.ds(r0, RPT)])
            plsc.subcore_barrier()

            if with_counts and p == 0:
                @pl.when(c == 0)
                def _():
                    start_counts(0, zc_v)
                    start_counts(1, zc_v)

            start_idx(0, 0)
            wait_idx(0)
            start_gather(0)
            start_idx(1, 1)

            def step(i, b, nb, first=False):
                wait_idx(nb)
                if not first:
                    wait_scatter(nb)
                start_gather(nb)
                wait_gather(b)
                start_scatter(b)
                start_idx(jnp.minimum(i + 2, NB - 1), b)

            step(0, 0, 1, first=True)

            def pair(k, carry):
                step(2 * k + 1, 1, 0)
                step(2 * k + 2, 0, 1)
                return carry

            lax.fori_loop(0, (NB - 3) // 2, pair, 0)
            step(NB - 2, 1, 0)
            wait_gather(0)
            start_scatter(0)
            wait_scatter(1)
            wait_scatter(0)
            wait_idx(1)
            if with_counts and p == 0:
                @pl.when(c == 0)
                def _():
                    wait_counts(0)
                    wait_counts(1)

            plsc.subcore_barrier()
            pltpu.sync_copy(acc_sh.at[pl.ds(r0, RPT)],
                            out_hbm.at[pl.ds(off + r0, RPT)])
            if with_counts and p == 0:
                @pl.when(c == 0)
                def _():
                    pltpu.sync_copy(cnt_sh.at[pl.ds(r0, RPT)],
                                    cnt_hbm.at[pl.ds(r0, RPT)])
            if p + 1 < n_passes:
                plsc.subcore_barrier()

    return pl.kernel(body, out_type=out_type, mesh=mesh, scratch_types=scratch)


R = 1280
GRID = NPAD // R


def _tc1_body(agg_ref, cnt_ref, x_ref, wl_ref, wr_ref, b_ref, out_ref):
    agg = jnp.concatenate([agg_ref[0], agg_ref[1]], axis=-1)
    inv = 1.0 / jnp.maximum(cnt_ref[...], 1.0)
    h = jnp.dot(agg * inv, wl_ref[...], preferred_element_type=jnp.float32)
    h += jnp.dot(x_ref[...], wr_ref[...], preferred_element_type=jnp.float32)
    h += b_ref[...]
    h = jnp.maximum(h, 0.0)
    for ch in range(4):
        out_ref[ch] = h[:, 128 * ch:128 * (ch + 1)]


def _tc2_body(agg_ref, cnt_ref, h1_ref, wl_ref, wr_ref, b_ref, wfc_ref,
              bfc_ref, out_ref):
    agg = jnp.concatenate([agg_ref[i] for i in range(4)], axis=-1)
    h1 = jnp.concatenate([h1_ref[i] for i in range(4)], axis=-1)
    inv = 1.0 / jnp.maximum(cnt_ref[...], 1.0)
    h = jnp.dot(agg * inv, wl_ref[...], preferred_element_type=jnp.float32)
    h += jnp.dot(h1, wr_ref[...], preferred_element_type=jnp.float32)
    h += b_ref[...]
    h = jnp.maximum(h, 0.0)
    out_ref[...] = jnp.dot(h, wfc_ref[...],
                           preferred_element_type=jnp.float32) + bfc_ref[...]


def _full(shape):
    return pl.BlockSpec(shape, lambda i: tuple(0 for _ in shape))


_tc1 = pl.pallas_call(
    _tc1_body,
    grid=(GRID,),
    in_specs=[
        pl.BlockSpec((2, R, 128), lambda i: (0, i, 0)),
        pl.BlockSpec((R, 1), lambda i: (i, 0)),
        pl.BlockSpec((R, D_IN), lambda i: (i, 0)),
        _full((D_IN, D_HID)),
        _full((D_IN, D_HID)),
        _full((1, D_HID)),
    ],
    out_specs=pl.BlockSpec((4, R, 128), lambda i: (0, i, 0)),
    out_shape=jax.ShapeDtypeStruct((4, NPAD, 128), jnp.float32),
)

_tc2 = pl.pallas_call(
    _tc2_body,
    grid=(GRID,),
    in_specs=[
        pl.BlockSpec((4, R, 128), lambda i: (0, i, 0)),
        pl.BlockSpec((R, 1), lambda i: (i, 0)),
        pl.BlockSpec((4, R, 128), lambda i: (0, i, 0)),
        _full((D_HID, D_HID)),
        _full((D_HID, D_HID)),
        _full((1, D_HID)),
        _full((D_HID, D_OUT)),
        _full((1, D_OUT)),
    ],
    out_specs=pl.BlockSpec((R, D_OUT), lambda i: (i, 0)),
    out_shape=jax.ShapeDtypeStruct((NPAD, D_OUT), jnp.float32),
)

_sc_agg2 = _make_sc_agg(2, with_counts=True)
_sc_agg4 = _make_sc_agg(4, with_counts=False)


def kernel(x, edge_index, W1l, b1, W1r, W2l, b2, W2r, Wfc, bfc):
    src = edge_index[0].astype(jnp.int32)
    dst = edge_index[1].astype(jnp.int32)
    xp = jnp.pad(x, ((0, NPAD - N), (0, 0)))
    xc = xp.reshape(NPAD, 2, 128).transpose(1, 0, 2).reshape(2 * NPAD, 128)
    z2d = jnp.zeros((RPT, 128), jnp.float32)
    z1d = jnp.zeros((RPT,), jnp.float32)

    agg1, cnt = _sc_agg2(src, dst, xc, z2d, z1d)
    cnt2 = cnt.reshape(NPAD, 1)
    h1c = _tc1(agg1.reshape(2, NPAD, 128), cnt2, xp,
               W1l.T, W1r.T, b1.reshape(1, D_HID))
    agg2, = _sc_agg4(src, dst, h1c.reshape(4 * NPAD, 128), z2d, z1d)
    outp = _tc2(agg2.reshape(4, NPAD, 128), cnt2, h1c,
                W2l.T, W2r.T, b2.reshape(1, D_HID),
                Wfc.T, bfc.reshape(1, D_OUT))
    return outp[:N]
